# baseline (device time: 15691 ns/iter reference)
import jax
import jax.numpy as jnp
from jax import lax
from jax.experimental import pallas as pl
from jax.experimental.pallas import tpu as pltpu

N_DEV = 4
N_LAYERS = 3
OFFSETS = (2, 1, 3)


def kernel(x, Win0, Wout0, Win1, Wout1, Win2, Wout2):
    b, d_model = x.shape
    rows = b // N_DEV

    def body(x_ref, win0_ref, wout0_ref, win1_ref, wout1_ref, win2_ref,
             wout2_ref, out_ref, part_ref, comm_ref, comm2_ref,
             x_v, wins_v, wouts_v, winb_ref, woutb_ref,
             w_sems, send_sems, recv_sems):
        my_pos = lax.axis_index("i")

        x_copy = pltpu.make_async_copy(x_ref, x_v, w_sems.at[6])
        x_copy.start()
        w_copies = []
        for r, (win, wout) in enumerate(
                [(win0_ref, wout0_ref), (win1_ref, wout1_ref),
                 (win2_ref, wout2_ref)]):
            cin = pltpu.make_async_copy(win, wins_v.at[r], w_sems.at[2 * r])
            cout = pltpu.make_async_copy(wout, wouts_v.at[r],
                                         w_sems.at[2 * r + 1])
            cin.start()
            cout.start()
            w_copies.append((cin, cout))

        barrier_sem = pltpu.get_barrier_semaphore()
        for d in range(1, N_DEV):
            pl.semaphore_signal(
                barrier_sem, inc=1,
                device_id=((my_pos + d) % N_DEV,),
                device_id_type=pl.DeviceIdType.MESH,
            )

        x_copy.wait()
        xb = x_v[...].astype(jnp.bfloat16)
        w_copies[0][0].wait()
        winb_ref[0] = wins_v[0].astype(jnp.bfloat16)
        w_copies[0][1].wait()
        woutb_ref[0] = wouts_v[0].astype(jnp.bfloat16)

        all_rdmas = []
        for r in range(N_LAYERS):
            h = jnp.dot(xb, winb_ref[r],
                        preferred_element_type=jnp.float32)
            h = jnp.maximum(h, 0.0).astype(jnp.bfloat16)
            part = jnp.dot(h, woutb_ref[r],
                           preferred_element_type=jnp.float32)
            part_ref[r] = part.astype(jnp.bfloat16)

            if r == 0:
                pl.semaphore_wait(barrier_sem, N_DEV - 1)

            if r < N_LAYERS - 1:
                rdma_by_d = {}
                for d in OFFSETS:
                    rdma = pltpu.make_async_remote_copy(
                        src_ref=part_ref.at[r],
                        dst_ref=comm_ref.at[r, 3 - d],
                        send_sem=send_sems.at[r, d - 1],
                        recv_sem=recv_sems.at[r, 3 - d],
                        device_id=((my_pos + d) % N_DEV,),
                        device_id_type=pl.DeviceIdType.MESH,
                    )
                    rdma.start()
                    rdma_by_d[d] = rdma
                all_rdmas.extend(rdma_by_d.values())
                w_copies[r + 1][0].wait()
                winb_ref[r + 1] = wins_v[r + 1].astype(jnp.bfloat16)
                w_copies[r + 1][1].wait()
                woutb_ref[r + 1] = wouts_v[r + 1].astype(jnp.bfloat16)
                acc = part
                for d in (1, 3, 2):
                    rdma_by_d[d].wait_recv()
                    acc = acc + comm_ref[r, 3 - d].astype(jnp.float32)
                xb = acc.astype(jnp.bfloat16)
            else:
                rdma_by_d = {}
                for d in OFFSETS:
                    dst = (my_pos + d) % N_DEV
                    rdma = pltpu.make_async_remote_copy(
                        src_ref=part_ref.at[r, pl.ds(dst * rows, rows)],
                        dst_ref=comm2_ref.at[3 - d],
                        send_sem=send_sems.at[r, d - 1],
                        recv_sem=recv_sems.at[r, 3 - d],
                        device_id=(dst,),
                        device_id_type=pl.DeviceIdType.MESH,
                    )
                    rdma.start()
                    rdma_by_d[d] = rdma
                all_rdmas.extend(rdma_by_d.values())
                out = part_ref[r, pl.ds(my_pos * rows, rows)].astype(jnp.float32)
                for d in (1, 3, 2):
                    rdma_by_d[d].wait_recv()
                    out = out + comm2_ref[3 - d].astype(jnp.float32)
                out_ref[...] = out

        for rdma in all_rdmas:
            rdma.wait_send()

    return pl.pallas_call(
        body,
        out_shape=jax.ShapeDtypeStruct((rows, d_model), jnp.float32),
        in_specs=[pl.BlockSpec(memory_space=pltpu.MemorySpace.HBM)] * 7,
        out_specs=pl.BlockSpec(memory_space=pltpu.VMEM),
        scratch_shapes=[
            pltpu.VMEM((N_LAYERS, b, d_model), jnp.bfloat16),
            pltpu.VMEM((N_LAYERS - 1, N_DEV - 1, b, d_model), jnp.bfloat16),
            pltpu.VMEM((N_DEV - 1, rows, d_model), jnp.bfloat16),
            pltpu.VMEM((b, d_model), jnp.float32),
            pltpu.VMEM((N_LAYERS,) + Win0.shape, jnp.float32),
            pltpu.VMEM((N_LAYERS,) + Wout0.shape, jnp.float32),
            pltpu.VMEM((N_LAYERS,) + Win0.shape, jnp.bfloat16),
            pltpu.VMEM((N_LAYERS,) + Wout0.shape, jnp.bfloat16),
            pltpu.SemaphoreType.DMA((2 * N_LAYERS + 1,)),
            pltpu.SemaphoreType.DMA((N_LAYERS, N_DEV - 1)),
            pltpu.SemaphoreType.DMA((N_LAYERS, N_DEV - 1)),
        ],
        compiler_params=pltpu.CompilerParams(collective_id=0),
    )(*[
        pltpu.with_memory_space_constraint(a, pltpu.MemorySpace.HBM)
        for a in (x, Win0, Wout0, Win1, Wout1, Win2, Wout2)
    ])


# device time: 15565 ns/iter; 1.0081x vs baseline; 1.0081x over previous
import jax
import jax.numpy as jnp
from jax import lax
from jax.experimental import pallas as pl
from jax.experimental.pallas import tpu as pltpu

N_DEV = 4
N_LAYERS = 3
OFFSETS = (2, 1, 3)
WAIT_ORDER = (1, 3, 2)


def kernel(x, Win0, Wout0, Win1, Wout1, Win2, Wout2):
    b, d_model = x.shape
    rows = b // N_DEV
    half = b // 2

    def body(x_ref, win0_ref, wout0_ref, win1_ref, wout1_ref, win2_ref,
             wout2_ref, out_ref, part_ref, comm_ref, comm2_ref,
             x_v, wins_v, wouts_v, winb_ref, woutb_ref,
             w_sems, send_sems, recv_sems):
        my_pos = lax.axis_index("i")

        x_copy = pltpu.make_async_copy(x_ref, x_v, w_sems.at[6])
        x_copy.start()
        w_copies = []
        for r, (win, wout) in enumerate(
                [(win0_ref, wout0_ref), (win1_ref, wout1_ref),
                 (win2_ref, wout2_ref)]):
            cin = pltpu.make_async_copy(win, wins_v.at[r], w_sems.at[2 * r])
            cout = pltpu.make_async_copy(wout, wouts_v.at[r],
                                         w_sems.at[2 * r + 1])
            cin.start()
            cout.start()
            w_copies.append((cin, cout))

        barrier_sem = pltpu.get_barrier_semaphore()
        for d in range(1, N_DEV):
            pl.semaphore_signal(
                barrier_sem, inc=1,
                device_id=((my_pos + d) % N_DEV,),
                device_id_type=pl.DeviceIdType.MESH,
            )

        x_copy.wait()
        xb = x_v[...].astype(jnp.bfloat16)
        w_copies[0][0].wait()
        winb_ref[0] = wins_v[0].astype(jnp.bfloat16)
        w_copies[0][1].wait()
        woutb_ref[0] = wouts_v[0].astype(jnp.bfloat16)

        def compute_half(r, xb_h):
            h = jnp.dot(xb_h, winb_ref[r], preferred_element_type=jnp.float32)
            h = jnp.maximum(h, 0.0).astype(jnp.bfloat16)
            return jnp.dot(h, woutb_ref[r], preferred_element_type=jnp.float32)

        rdma = {}
        all_rdmas = []

        def send_half(r, s):
            for d in OFFSETS:
                op = pltpu.make_async_remote_copy(
                    src_ref=part_ref.at[r, pl.ds(s * half, half)],
                    dst_ref=comm_ref.at[r, 3 - d, pl.ds(s * half, half)],
                    send_sem=send_sems.at[r, d - 1, s],
                    recv_sem=recv_sems.at[r, 3 - d, s],
                    device_id=((my_pos + d) % N_DEV,),
                    device_id_type=pl.DeviceIdType.MESH,
                )
                op.start()
                rdma[(r, s, d)] = op
                all_rdmas.append(op)

        def cast_weights(r):
            w_copies[r][0].wait()
            winb_ref[r] = wins_v[r].astype(jnp.bfloat16)
            w_copies[r][1].wait()
            woutb_ref[r] = wouts_v[r].astype(jnp.bfloat16)

        part = {}
        part[0] = compute_half(0, xb[:half])
        part_ref[0, pl.ds(0, half)] = part[0].astype(jnp.bfloat16)
        pl.semaphore_wait(barrier_sem, N_DEV - 1)
        send_half(0, 0)
        part[1] = compute_half(0, xb[half:])
        part_ref[0, pl.ds(half, half)] = part[1].astype(jnp.bfloat16)
        send_half(0, 1)
        cast_weights(1)

        for r in (1, 2):
            for s in (0, 1):
                acc = part[s]
                for d in WAIT_ORDER:
                    rdma[(r - 1, s, d)].wait_recv()
                    acc = acc + comm_ref[
                        r - 1, 3 - d, pl.ds(s * half, half)].astype(jnp.float32)
                part[s] = compute_half(r, acc.astype(jnp.bfloat16))
                part_ref[r, pl.ds(s * half, half)] = part[s].astype(jnp.bfloat16)
                if r < N_LAYERS - 1:
                    send_half(r, s)
            if r == 1:
                cast_weights(2)

        rs = {}
        for d in OFFSETS:
            dst = (my_pos + d) % N_DEV
            op = pltpu.make_async_remote_copy(
                src_ref=part_ref.at[2, pl.ds(dst * rows, rows)],
                dst_ref=comm2_ref.at[3 - d],
                send_sem=send_sems.at[2, d - 1, 0],
                recv_sem=recv_sems.at[2, 3 - d, 0],
                device_id=(dst,),
                device_id_type=pl.DeviceIdType.MESH,
            )
            op.start()
            rs[d] = op
            all_rdmas.append(op)
        out = part_ref[2, pl.ds(my_pos * rows, rows)].astype(jnp.float32)
        for d in WAIT_ORDER:
            rs[d].wait_recv()
            out = out + comm2_ref[3 - d].astype(jnp.float32)
        out_ref[...] = out

        for op in all_rdmas:
            op.wait_send()

    return pl.pallas_call(
        body,
        out_shape=jax.ShapeDtypeStruct((rows, d_model), jnp.float32),
        in_specs=[pl.BlockSpec(memory_space=pltpu.MemorySpace.HBM)] * 7,
        out_specs=pl.BlockSpec(memory_space=pltpu.VMEM),
        scratch_shapes=[
            pltpu.VMEM((N_LAYERS, b, d_model), jnp.bfloat16),
            pltpu.VMEM((N_LAYERS - 1, N_DEV - 1, b, d_model), jnp.bfloat16),
            pltpu.VMEM((N_DEV - 1, rows, d_model), jnp.bfloat16),
            pltpu.VMEM((b, d_model), jnp.float32),
            pltpu.VMEM((N_LAYERS,) + Win0.shape, jnp.float32),
            pltpu.VMEM((N_LAYERS,) + Wout0.shape, jnp.float32),
            pltpu.VMEM((N_LAYERS,) + Win0.shape, jnp.bfloat16),
            pltpu.VMEM((N_LAYERS,) + Wout0.shape, jnp.bfloat16),
            pltpu.SemaphoreType.DMA((2 * N_LAYERS + 1,)),
            pltpu.SemaphoreType.DMA((N_LAYERS, N_DEV - 1, 2)),
            pltpu.SemaphoreType.DMA((N_LAYERS, N_DEV - 1, 2)),
        ],
        compiler_params=pltpu.CompilerParams(collective_id=0),
    )(*[
        pltpu.with_memory_space_constraint(a, pltpu.MemorySpace.HBM)
        for a in (x, Win0, Wout0, Win1, Wout1, Win2, Wout2)
    ])
